# baseline (speedup 1.0000x reference)
import jax
import jax.numpy as jnp
from jax import lax
from jax.experimental import pallas as pl

BATCH = 4096
D_IN = 4096
K_RAW = 7000
KP = 7040
CD = 32
NQ = 3
BM = 128


def _body(x_ref, w0, b0, w1, b1, w2, b2, dw0, db0, dw1, db1, dw2, db2,
          cbc_ref, cbt_ref, recon_ref, idx0_ref, idx1_ref, idx2_ref, loss_ref):
    i = pl.program_id(0)
    x = x_ref[...]
    h = jnp.maximum(jnp.dot(x, w0[...], preferred_element_type=jnp.float32) + b0[...], 0.0)
    h = jnp.maximum(jnp.dot(h, w1[...], preferred_element_type=jnp.float32) + b1[...], 0.0)
    z = jnp.dot(h, w2[...], preferred_element_type=jnp.float32) + b2[...]

    lane1 = lax.broadcasted_iota(jnp.int32, (1, KP), 1)
    lane2 = lax.broadcasted_iota(jnp.int32, (BM, KP), 1)
    r = z
    idxs = []
    qsts = []
    lsums = []
    for q in range(NQ):
        cbt = cbt_ref[q]
        n = jnp.sum(cbt * cbt, axis=0, keepdims=True)
        n = jnp.where(lane1 < K_RAW, n, jnp.float32(1e30))
        c = jnp.sum(r * r, axis=-1, keepdims=True)
        m = jnp.dot(r, cbt, preferred_element_type=jnp.float32)
        dist = (c - 2.0 * m) + n
        mn = jnp.min(dist, axis=-1, keepdims=True)
        idx = jnp.min(jnp.where(dist == mn, lane2, KP), axis=-1)
        lT = jnp.broadcast_to((idx % 128)[None, :], (CD, BM))
        cT = jnp.broadcast_to((idx // 128)[None, :], (CD, BM))

        def gbody(ch, acc):
            chunk = cbc_ref[q, ch]
            g = jnp.take_along_axis(chunk, lT, axis=1)
            return jnp.where(cT == ch, g, acc)

        acc = lax.fori_loop(0, KP // 128, gbody, jnp.zeros((CD, BM), jnp.float32))
        quant = acc.T
        qst = r + (quant - r)
        r = r - quant
        lsums.append(jnp.sum(r * r))
        idxs.append(idx)
        qsts.append(qst)

    zq = (qsts[0] + qsts[1]) + qsts[2]
    g = jnp.maximum(jnp.dot(zq, dw0[...], preferred_element_type=jnp.float32) + db0[...], 0.0)
    g = jnp.maximum(jnp.dot(g, dw1[...], preferred_element_type=jnp.float32) + db1[...], 0.0)
    recon_ref[...] = jnp.dot(g, dw2[...], preferred_element_type=jnp.float32) + db2[...]
    idx0_ref[...] = idxs[0]
    idx1_ref[...] = idxs[1]
    idx2_ref[...] = idxs[2]

    li = lax.broadcasted_iota(jnp.int32, (8, 128), 1)
    part = (jnp.where(li == 0, lsums[0], 0.0)
            + jnp.where(li == 1, lsums[1], 0.0)
            + jnp.where(li == 2, lsums[2], 0.0))

    @pl.when(i == 0)
    def _init():
        loss_ref[...] = jnp.zeros_like(loss_ref)

    loss_ref[...] += part


def kernel(x, enc_W0, enc_b0, enc_W1, enc_b1, enc_W2, enc_b2,
           dec_W0, dec_b0, dec_W1, dec_b1, dec_W2, dec_b2, codebooks):
    cb_pad = jnp.pad(codebooks, ((0, 0), (0, KP - K_RAW), (0, 0)))
    cbt = jnp.transpose(cb_pad, (0, 2, 1))
    cbc = jnp.transpose(cb_pad.reshape(NQ, KP // 128, 128, CD), (0, 1, 3, 2))
    row = lambda v: v.reshape(1, -1)

    grid = (BATCH // BM,)
    full = lambda a: pl.BlockSpec(a.shape, lambda i: (0,) * a.ndim)
    in_specs = [pl.BlockSpec((BM, D_IN), lambda i: (i, 0))]
    weights = [enc_W0, row(enc_b0), enc_W1, row(enc_b1), enc_W2, row(enc_b2),
               dec_W0, row(dec_b0), dec_W1, row(dec_b1), dec_W2, row(dec_b2),
               cbc, cbt]
    in_specs += [full(w) for w in weights]

    out_shapes = [
        jax.ShapeDtypeStruct((BATCH, D_IN), jnp.float32),
        jax.ShapeDtypeStruct((BATCH,), jnp.int32),
        jax.ShapeDtypeStruct((BATCH,), jnp.int32),
        jax.ShapeDtypeStruct((BATCH,), jnp.int32),
        jax.ShapeDtypeStruct((8, 128), jnp.float32),
    ]
    out_specs = [
        pl.BlockSpec((BM, D_IN), lambda i: (i, 0)),
        pl.BlockSpec((BM,), lambda i: (i,)),
        pl.BlockSpec((BM,), lambda i: (i,)),
        pl.BlockSpec((BM,), lambda i: (i,)),
        pl.BlockSpec((8, 128), lambda i: (0, 0)),
    ]

    recon, i0, i1, i2, lossbuf = pl.pallas_call(
        _body,
        grid=grid,
        in_specs=in_specs,
        out_specs=out_specs,
        out_shape=out_shapes,
    )(x, *weights)

    indices = jnp.stack([i0, i1, i2], axis=-1)
    commit_loss = lossbuf[0, :NQ] * jnp.float32(1.0 / (BATCH * CD))
    return recon, indices, commit_loss


# speedup vs baseline: 18.1675x; 18.1675x over previous
import jax
import jax.numpy as jnp
from jax import lax
from jax.experimental import pallas as pl

BATCH = 4096
D_IN = 4096
K_RAW = 7000
KP = 7040
CD = 32
NQ = 3
BM = 128


def _body(x_ref, w0, b0, w1, b1, w2, b2, dw0, db0, dw1, db1, dw2, db2,
          cb3_ref, cbt2_ref, recon_ref, idx0_ref, idx1_ref, idx2_ref, loss_ref):
    i = pl.program_id(0)
    x = x_ref[...]
    h = jnp.maximum(jnp.dot(x, w0[...], preferred_element_type=jnp.float32) + b0[...], 0.0)
    h = jnp.maximum(jnp.dot(h, w1[...], preferred_element_type=jnp.float32) + b1[...], 0.0)
    z = jnp.dot(h, w2[...], preferred_element_type=jnp.float32) + b2[...]

    lane1 = lax.broadcasted_iota(jnp.int32, (1, KP), 1)
    lane2 = lax.broadcasted_iota(jnp.int32, (BM, KP), 1)
    r = z
    idxs = []
    qsts = []
    lsums = []
    for q in range(NQ):
        cbt2 = cbt2_ref[q]
        n = jnp.sum(cbt2 * cbt2, axis=0, keepdims=True) * jnp.float32(0.25)
        n = jnp.where(lane1 < K_RAW, n, jnp.float32(1e30))
        c = jnp.sum(r * r, axis=-1, keepdims=True)
        m2 = jnp.dot(r, cbt2, preferred_element_type=jnp.float32)
        dist = (c + m2) + n
        mn = jnp.min(dist, axis=-1, keepdims=True)
        idx = jnp.min(jnp.where(dist == mn, lane2, KP), axis=-1)
        oh = (lane2 == idx[:, None]).astype(jnp.bfloat16)
        q3 = jnp.dot(oh, cb3_ref[q], preferred_element_type=jnp.float32)
        quant = (q3[:, 0:CD] + q3[:, CD:2 * CD]) + q3[:, 2 * CD:3 * CD]
        qst = r + (quant - r)
        r = r - quant
        lsums.append(jnp.sum(r * r))
        idxs.append(idx)
        qsts.append(qst)

    zq = (qsts[0] + qsts[1]) + qsts[2]
    g = jnp.maximum(jnp.dot(zq, dw0[...], preferred_element_type=jnp.float32) + db0[...], 0.0)
    g = jnp.maximum(jnp.dot(g, dw1[...], preferred_element_type=jnp.float32) + db1[...], 0.0)
    recon_ref[...] = jnp.dot(g, dw2[...], preferred_element_type=jnp.float32) + db2[...]
    idx0_ref[...] = idxs[0]
    idx1_ref[...] = idxs[1]
    idx2_ref[...] = idxs[2]

    li = lax.broadcasted_iota(jnp.int32, (8, 128), 1)
    part = (jnp.where(li == 0, lsums[0], 0.0)
            + jnp.where(li == 1, lsums[1], 0.0)
            + jnp.where(li == 2, lsums[2], 0.0))

    @pl.when(i == 0)
    def _init():
        loss_ref[...] = jnp.zeros_like(loss_ref)

    loss_ref[...] += part


def kernel(x, enc_W0, enc_b0, enc_W1, enc_b1, enc_W2, enc_b2,
           dec_W0, dec_b0, dec_W1, dec_b1, dec_W2, dec_b2, codebooks):
    cb_pad = jnp.pad(codebooks, ((0, 0), (0, KP - K_RAW), (0, 0)))
    cbt2 = jnp.transpose(cb_pad, (0, 2, 1)) * jnp.float32(-2.0)
    hi = cb_pad.astype(jnp.bfloat16)
    rem = cb_pad - hi.astype(jnp.float32)
    mid = rem.astype(jnp.bfloat16)
    lo = (rem - mid.astype(jnp.float32)).astype(jnp.bfloat16)
    cb3 = jnp.concatenate([hi, mid, lo], axis=-1)
    row = lambda v: v.reshape(1, -1)

    grid = (BATCH // BM,)
    full = lambda a: pl.BlockSpec(a.shape, lambda i: (0,) * a.ndim)
    in_specs = [pl.BlockSpec((BM, D_IN), lambda i: (i, 0))]
    weights = [enc_W0, row(enc_b0), enc_W1, row(enc_b1), enc_W2, row(enc_b2),
               dec_W0, row(dec_b0), dec_W1, row(dec_b1), dec_W2, row(dec_b2),
               cb3, cbt2]
    in_specs += [full(w) for w in weights]

    out_shapes = [
        jax.ShapeDtypeStruct((BATCH, D_IN), jnp.float32),
        jax.ShapeDtypeStruct((BATCH,), jnp.int32),
        jax.ShapeDtypeStruct((BATCH,), jnp.int32),
        jax.ShapeDtypeStruct((BATCH,), jnp.int32),
        jax.ShapeDtypeStruct((8, 128), jnp.float32),
    ]
    out_specs = [
        pl.BlockSpec((BM, D_IN), lambda i: (i, 0)),
        pl.BlockSpec((BM,), lambda i: (i,)),
        pl.BlockSpec((BM,), lambda i: (i,)),
        pl.BlockSpec((BM,), lambda i: (i,)),
        pl.BlockSpec((8, 128), lambda i: (0, 0)),
    ]

    recon, i0, i1, i2, lossbuf = pl.pallas_call(
        _body,
        grid=grid,
        in_specs=in_specs,
        out_specs=out_specs,
        out_shape=out_shapes,
    )(x, *weights)

    indices = jnp.stack([i0, i1, i2], axis=-1)
    commit_loss = lossbuf[0, :NQ] * jnp.float32(1.0 / (BATCH * CD))
    return recon, indices, commit_loss
